# baseline (device time: 37098 ns/iter reference)
import jax
import jax.numpy as jnp
from jax import lax
from jax.experimental import pallas as pl
from jax.experimental.pallas import tpu as pltpu

B, S, H, Dh, Dr = 2, 256, 16, 64, 32
D = 1024
BS = B * S
DC_SH = 64
PACK = 3 * DC_SH


def _dot(a, b, trans_b=False):
    dn = (((1,), (1 if trans_b else 0,)), ((), ()))
    return lax.dot_general(a, b, dn, preferred_element_type=jnp.float32)


def kernel(x, Wdkv, Wuk, Wuv, Wq, Wqr, Wkr, Wo):
    def body(x_ref, wdkv_ref, wuk_ref, wuv_ref, wq_ref, wqr_ref, wkr_ref,
             wo_ref, out_ref, pack_snd, pack_rcv,
             x_v, wq_v, wqr_v, wkr_v, wo_v,
             send_sems, recv_sems, cp_sems):
        my_x = lax.axis_index("x")
        my_y = lax.axis_index("y")
        y_nbr = (my_x, 1 - my_y)

        barrier = pltpu.get_barrier_semaphore()
        pl.semaphore_signal(barrier, inc=1, device_id=y_nbr,
                            device_id_type=pl.DeviceIdType.MESH)
        pl.semaphore_wait(barrier, 1)

        pack_snd[0:DC_SH] = jnp.transpose(wdkv_ref[...])
        pack_snd[DC_SH:2 * DC_SH] = wuk_ref[...]
        pack_snd[2 * DC_SH:PACK] = wuv_ref[...]
        y_rdma = pltpu.make_async_remote_copy(
            src_ref=pack_snd, dst_ref=pack_rcv,
            send_sem=send_sems.at[0], recv_sem=recv_sems.at[0],
            device_id=y_nbr, device_id_type=pl.DeviceIdType.MESH)
        y_rdma.start()

        cps = []
        for i, (src, dst) in enumerate(
                [(x_ref, x_v), (wq_ref, wq_v), (wqr_ref, wqr_v),
                 (wkr_ref, wkr_v), (wo_ref, wo_v)]):
            cp = pltpu.make_async_copy(src, dst, cp_sems.at[i])
            cp.start()
            cps.append(cp)

        cps[0].wait()
        xf = x_v[...].reshape(BS, D)
        c1 = _dot(xf, wdkv_ref[...])
        K = _dot(c1, wuk_ref[...])
        V = _dot(c1, wuv_ref[...])
        cps[1].wait()
        Q = _dot(xf, wq_v[...])
        cps[2].wait()
        Qr = _dot(xf, wqr_v[...])
        cps[3].wait()
        Kr = _dot(xf, wkr_v[...])

        y_rdma.wait()
        c2 = _dot(xf, pack_rcv[0:DC_SH], trans_b=True)
        K = K + _dot(c2, pack_rcv[DC_SH:2 * DC_SH])
        V = V + _dot(c2, pack_rcv[2 * DC_SH:PACK])

        scale = (Dh + Dr) ** -0.5
        o_parts = []
        for b in range(B):
            row = slice(b * S, (b + 1) * S)
            Kr_b = Kr[row]
            for h in range(H):
                qh = Q[row, h * Dh:(h + 1) * Dh]
                kh = K[row, h * Dh:(h + 1) * Dh]
                vh = V[row, h * Dh:(h + 1) * Dh]
                qrh = Qr[row, h * Dr:(h + 1) * Dr]
                s = (_dot(qh, kh, trans_b=True)
                     + _dot(qrh, Kr_b, trans_b=True)) * scale
                p = jnp.exp(s)
                denom = jnp.sum(p, axis=-1, keepdims=True)
                o_parts.append(_dot(p, vh) / denom)
        O = jnp.concatenate(
            [jnp.concatenate(o_parts[b * H:(b + 1) * H], axis=-1)
             for b in range(B)], axis=0)
        cps[4].wait()
        out = _dot(O, wo_v[...])
        out_ref[...] = out.reshape(B, S, D)

    return pl.pallas_call(
        body,
        out_shape=jax.ShapeDtypeStruct((B, S, D), jnp.float32),
        in_specs=[
            pl.BlockSpec(memory_space=pl.ANY),
            pl.BlockSpec(memory_space=pltpu.VMEM),
            pl.BlockSpec(memory_space=pltpu.VMEM),
            pl.BlockSpec(memory_space=pltpu.VMEM),
            pl.BlockSpec(memory_space=pl.ANY),
            pl.BlockSpec(memory_space=pl.ANY),
            pl.BlockSpec(memory_space=pl.ANY),
            pl.BlockSpec(memory_space=pl.ANY),
        ],
        out_specs=pl.BlockSpec(memory_space=pltpu.VMEM),
        scratch_shapes=[
            pltpu.VMEM((PACK, D), jnp.float32),
            pltpu.VMEM((PACK, D), jnp.float32),
            pltpu.VMEM((B, S, D), jnp.float32),
            pltpu.VMEM((D, D), jnp.float32),
            pltpu.VMEM((D, H * Dr), jnp.float32),
            pltpu.VMEM((D, Dr), jnp.float32),
            pltpu.VMEM((D, D), jnp.float32),
            pltpu.SemaphoreType.DMA((1,)),
            pltpu.SemaphoreType.DMA((1,)),
            pltpu.SemaphoreType.DMA((5,)),
        ],
        compiler_params=pltpu.CompilerParams(collective_id=0),
    )(x, Wdkv, Wuk, Wuv, Wq, Wqr, Wkr, Wo)


# device time: 34622 ns/iter; 1.0715x vs baseline; 1.0715x over previous
import jax
import jax.numpy as jnp
from jax import lax
from jax.experimental import pallas as pl
from jax.experimental.pallas import tpu as pltpu

B, S, H, Dh, Dr = 2, 256, 16, 64, 32
D = 1024
BS = B * S
DC_SH = 64
PACK = 3 * DC_SH


def _dot(a, b, trans_b=False):
    dn = (((1,), (1 if trans_b else 0,)), ((), ()))
    return lax.dot_general(a, b, dn, preferred_element_type=jnp.float32)


def kernel(x, Wdkv, Wuk, Wuv, Wq, Wqr, Wkr, Wo):
    def body(x_ref, wdkv_ref, wuk_ref, wuv_ref, wq_ref, wqr_ref, wkr_ref,
             wo_ref, out_ref, pack_snd, pack_rcv,
             x_v, wq_v, wqr_v, wkr_v, wo_v,
             send_sems, recv_sems, cp_sems):
        my_x = lax.axis_index("x")
        my_y = lax.axis_index("y")
        y_nbr = (my_x, 1 - my_y)

        barrier = pltpu.get_barrier_semaphore()
        pl.semaphore_signal(barrier, inc=1, device_id=y_nbr,
                            device_id_type=pl.DeviceIdType.MESH)
        pl.semaphore_wait(barrier, 1)

        pack_snd[0:DC_SH] = jnp.transpose(wdkv_ref[...])
        pack_snd[DC_SH:2 * DC_SH] = wuk_ref[...]
        pack_snd[2 * DC_SH:PACK] = wuv_ref[...]
        y_rdma = pltpu.make_async_remote_copy(
            src_ref=pack_snd, dst_ref=pack_rcv,
            send_sem=send_sems.at[0], recv_sem=recv_sems.at[0],
            device_id=y_nbr, device_id_type=pl.DeviceIdType.MESH)
        y_rdma.start()

        cps = []
        for i, (src, dst) in enumerate(
                [(x_ref, x_v), (wq_ref, wq_v), (wqr_ref, wqr_v),
                 (wkr_ref, wkr_v), (wo_ref, wo_v)]):
            cp = pltpu.make_async_copy(src, dst, cp_sems.at[i])
            cp.start()
            cps.append(cp)

        cps[0].wait()
        xf = x_v[...].reshape(BS, D)
        c1 = _dot(xf, wdkv_ref[...])
        K = _dot(c1, wuk_ref[...])
        V = _dot(c1, wuv_ref[...])
        cps[1].wait()
        Q = _dot(xf, wq_v[...])
        cps[2].wait()
        Qr = _dot(xf, wqr_v[...])
        cps[3].wait()
        Kr = _dot(xf, wkr_v[...])

        y_rdma.wait()
        c2 = _dot(xf, pack_rcv[0:DC_SH], trans_b=True)
        K = K + _dot(c2, pack_rcv[DC_SH:2 * DC_SH])
        V = V + _dot(c2, pack_rcv[2 * DC_SH:PACK])

        scale = (Dh + Dr) ** -0.5
        o_parts = []
        for b in range(B):
            row = slice(b * S, (b + 1) * S)
            Kr_b = Kr[row]
            for h in range(H):
                qh = Q[row, h * Dh:(h + 1) * Dh]
                kh = K[row, h * Dh:(h + 1) * Dh]
                vh = V[row, h * Dh:(h + 1) * Dh]
                qrh = Qr[row, h * Dr:(h + 1) * Dr]
                qcat = jnp.concatenate([qh, qrh], axis=1)
                kcat = jnp.concatenate([kh, Kr_b], axis=1)
                s = _dot(qcat, kcat, trans_b=True) * scale
                p = jnp.exp(s)
                denom = jnp.sum(p, axis=-1, keepdims=True)
                o_parts.append(_dot(p, vh) / denom)
        O = jnp.concatenate(
            [jnp.concatenate(o_parts[b * H:(b + 1) * H], axis=-1)
             for b in range(B)], axis=0)
        cps[4].wait()
        out = _dot(O, wo_v[...])
        out_ref[...] = out.reshape(B, S, D)

    return pl.pallas_call(
        body,
        out_shape=jax.ShapeDtypeStruct((B, S, D), jnp.float32),
        in_specs=[
            pl.BlockSpec(memory_space=pl.ANY),
            pl.BlockSpec(memory_space=pltpu.VMEM),
            pl.BlockSpec(memory_space=pltpu.VMEM),
            pl.BlockSpec(memory_space=pltpu.VMEM),
            pl.BlockSpec(memory_space=pl.ANY),
            pl.BlockSpec(memory_space=pl.ANY),
            pl.BlockSpec(memory_space=pl.ANY),
            pl.BlockSpec(memory_space=pl.ANY),
        ],
        out_specs=pl.BlockSpec(memory_space=pltpu.VMEM),
        scratch_shapes=[
            pltpu.VMEM((PACK, D), jnp.float32),
            pltpu.VMEM((PACK, D), jnp.float32),
            pltpu.VMEM((B, S, D), jnp.float32),
            pltpu.VMEM((D, D), jnp.float32),
            pltpu.VMEM((D, H * Dr), jnp.float32),
            pltpu.VMEM((D, Dr), jnp.float32),
            pltpu.VMEM((D, D), jnp.float32),
            pltpu.SemaphoreType.DMA((1,)),
            pltpu.SemaphoreType.DMA((1,)),
            pltpu.SemaphoreType.DMA((5,)),
        ],
        compiler_params=pltpu.CompilerParams(collective_id=0),
    )(x, Wdkv, Wuk, Wuv, Wq, Wqr, Wkr, Wo)
